# Initial kernel scaffold; baseline (speedup 1.0000x reference)
import jax
import jax.numpy as jnp
from jax.experimental import pallas as pl


def kernel(info_cont_feat, info_cate_feat, home_cont_feat, home_cate_feat, away_cont_feat, away_cate_feat, emb_home, emb_away, emb_home_conf, emb_away_conf, emb_tour, emb_city, emb_country, W_home, b_home, W_away, b_away, W_info, b_info, W_joint, b_joint, W_fc1, b_fc1, W_out, b_out):
    raise NotImplementedError("write your pallas kernel here")


# speedup vs baseline: 1.5709x; 1.5709x over previous
import functools

import jax
import jax.numpy as jnp
from jax import lax
from jax.experimental import pallas as pl
from jax.experimental.pallas import tpu as pltpu
from jax.experimental.pallas import tpu_sc as plsc

NC, NS = 2, 16
NW = NC * NS
CHUNK = 128
LANES = 16


def _round8(n):
    return (n + 7) // 8 * 8


def kernel(info_cont_feat, info_cate_feat, home_cont_feat, home_cate_feat,
           away_cont_feat, away_cate_feat, emb_home, emb_away, emb_home_conf,
           emb_away_conf, emb_tour, emb_city, emb_country, W_home, b_home,
           W_away, b_away, W_info, b_info, W_joint, b_joint, W_fc1, b_fc1,
           W_out, b_out):
    B = info_cont_feat.shape[0]
    D = W_home.shape[1]
    E = emb_home.shape[1]
    CH = home_cont_feat.shape[1]
    CI = info_cont_feat.shape[1]

    tables = [emb_home, emb_home_conf, emb_away, emb_away_conf,
              emb_tour, emb_city, emb_country]
    sizes = [t.shape[0] for t in tables]
    padded = [_round8(s) for s in sizes]
    offs = []
    acc = 0
    for p in padded:
        offs.append(acc)
        acc += p
    RP = acc

    tables_p = [t if t.shape[0] == p else jnp.pad(t, ((0, p - t.shape[0]), (0, 0)))
                for t, p in zip(tables, padded)]

    def _proj_body(eh, ehc, ea, eac, et, eci, eco, wh, wa, wi, out_ref):
        regions = [
            (eh,  wh,  CH),
            (ehc, wh,  CH + E),
            (ea,  wa,  CH),
            (eac, wa,  CH + E),
            (et,  wi,  CI),
            (eci, wi,  CI + E),
            (eco, wi,  CI + 2 * E),
        ]
        for k, (eref, wref, ws) in enumerate(regions):
            out_ref[pl.ds(offs[k], padded[k]), :] = jnp.dot(
                eref[...], wref[pl.ds(ws, E), :],
                preferred_element_type=jnp.float32)

    ptab = pl.pallas_call(
        _proj_body,
        out_shape=jax.ShapeDtypeStruct((RP, D), jnp.float32),
    )(*tables_p, W_home, W_away, W_info)

    gidx = jnp.stack([
        home_cate_feat[:, 0] + offs[0],
        home_cate_feat[:, 1] + offs[1],
        away_cate_feat[:, 0] + offs[2],
        away_cate_feat[:, 1] + offs[3],
        info_cate_feat[:, 0] + offs[4],
        info_cate_feat[:, 1] + offs[5],
        info_cate_feat[:, 2] + offs[6],
    ], axis=0).astype(jnp.int32)
    nchunks = B // CHUNK
    idx_chunks = gidx.reshape(7, nchunks, CHUNK).transpose(1, 0, 2)

    nsub = nchunks // NW

    mesh = plsc.VectorSubcoreMesh(core_axis_name="c", subcore_axis_name="s",
                                  num_cores=NC, num_subcores=NS)

    @functools.partial(
        pl.kernel,
        out_type=(jax.ShapeDtypeStruct((B, D), jnp.float32),) * 3,
        mesh=mesh,
        scratch_types=[
            pltpu.VMEM((7, CHUNK), jnp.int32),
            [pltpu.VMEM((CHUNK, D), jnp.float32) for _ in range(7)],
            pltpu.SemaphoreType.DMA,
        ],
    )
    def _gather_sum(ptab_hbm, idx_hbm, gh_out, ga_out, gi_out,
                    idx_v, bufs, sem):
        wid = lax.axis_index("s") * NC + lax.axis_index("c")

        def do_chunk(s, carry):
            chunk = wid * nsub + s
            base = chunk * CHUNK
            pltpu.sync_copy(idx_hbm.at[chunk], idx_v)
            descs = [pltpu.async_copy(ptab_hbm.at[idx_v.at[t]], bufs[t], sem)
                     for t in range(7)]
            for d in descs:
                d.wait()

            def addrow(r, c):
                for cc in range(D // LANES):
                    sl = pl.ds(cc * LANES, LANES)
                    plsc.addupdate(bufs[0].at[r, sl], bufs[1][r, sl])
                    plsc.addupdate(bufs[2].at[r, sl], bufs[3][r, sl])
                    plsc.addupdate(bufs[4].at[r, sl],
                                   bufs[5][r, sl] + bufs[6][r, sl])
                return c

            lax.fori_loop(0, CHUNK, addrow, 0, unroll=False)
            pltpu.sync_copy(bufs[0], gh_out.at[pl.ds(base, CHUNK)])
            pltpu.sync_copy(bufs[2], ga_out.at[pl.ds(base, CHUNK)])
            pltpu.sync_copy(bufs[4], gi_out.at[pl.ds(base, CHUNK)])
            return carry

        lax.fori_loop(0, nsub, do_chunk, 0, unroll=False)

    g_home, g_away, g_info = _gather_sum(ptab, idx_chunks)

    BM = 1024
    grid = (B // BM,)

    def _mlp_body(hc, ac, ic, gh, ga, gi, wh, wa, wi, bh, ba, bi,
                  wj, bj, wf, bf, wo, bo, out_ref):
        h = jnp.maximum(
            jnp.dot(hc[...], wh[pl.ds(0, CH), :],
                    preferred_element_type=jnp.float32) + gh[...] + bh[...], 0.0)
        a = jnp.maximum(
            jnp.dot(ac[...], wa[pl.ds(0, CH), :],
                    preferred_element_type=jnp.float32) + ga[...] + ba[...], 0.0)
        i = jnp.maximum(
            jnp.dot(ic[...], wi[pl.ds(0, CI), :],
                    preferred_element_type=jnp.float32) + gi[...] + bi[...], 0.0)
        j = jnp.maximum(
            jnp.dot(h, wj[pl.ds(0, D), :], preferred_element_type=jnp.float32)
            + jnp.dot(a, wj[pl.ds(D, D), :], preferred_element_type=jnp.float32)
            + jnp.dot(i, wj[pl.ds(2 * D, D), :], preferred_element_type=jnp.float32)
            + bj[...], 0.0)
        f = jnp.maximum(
            jnp.dot(j, wf[...], preferred_element_type=jnp.float32) + bf[...], 0.0)
        logit = jnp.sum(f * wo[...], axis=1, keepdims=True) + bo[...]
        out_ref[...] = jax.nn.sigmoid(logit)

    def _rows(cols):
        return pl.BlockSpec((BM, cols), lambda i: (i, 0))

    def _whole(shape):
        return pl.BlockSpec(shape, lambda i: (0, 0))

    out = pl.pallas_call(
        _mlp_body,
        grid=grid,
        in_specs=[
            _rows(CH), _rows(CH), _rows(CI),
            _rows(D), _rows(D), _rows(D),
            _whole(W_home.shape), _whole(W_away.shape), _whole(W_info.shape),
            _whole((1, D)), _whole((1, D)), _whole((1, D)),
            _whole(W_joint.shape), _whole((1, D)),
            _whole(W_fc1.shape), _whole((1, D)),
            _whole((1, D)), _whole((1, 1)),
        ],
        out_specs=pl.BlockSpec((BM, 1), lambda i: (i, 0)),
        out_shape=jax.ShapeDtypeStruct((B, 1), jnp.float32),
        compiler_params=pltpu.CompilerParams(
            dimension_semantics=("arbitrary",)),
    )(home_cont_feat, away_cont_feat, info_cont_feat,
      g_home, g_away, g_info,
      W_home, W_away, W_info,
      b_home.reshape(1, D), b_away.reshape(1, D), b_info.reshape(1, D),
      W_joint, b_joint.reshape(1, D),
      W_fc1, b_fc1.reshape(1, D),
      W_out.reshape(1, D), b_out.reshape(1, 1))
    return out


# speedup vs baseline: 2.3642x; 1.5050x over previous
import functools

import jax
import jax.numpy as jnp
from jax import lax
from jax.experimental import pallas as pl
from jax.experimental.pallas import tpu as pltpu
from jax.experimental.pallas import tpu_sc as plsc

NC, NS = 2, 16
NW = NC * NS
CHUNK = 64
LANES = 16


def _round8(n):
    return (n + 7) // 8 * 8


def kernel(info_cont_feat, info_cate_feat, home_cont_feat, home_cate_feat,
           away_cont_feat, away_cate_feat, emb_home, emb_away, emb_home_conf,
           emb_away_conf, emb_tour, emb_city, emb_country, W_home, b_home,
           W_away, b_away, W_info, b_info, W_joint, b_joint, W_fc1, b_fc1,
           W_out, b_out):
    B = info_cont_feat.shape[0]
    D = W_home.shape[1]
    E = emb_home.shape[1]
    CH = home_cont_feat.shape[1]
    CI = info_cont_feat.shape[1]

    tables = [emb_home, emb_home_conf, emb_away, emb_away_conf,
              emb_tour, emb_city, emb_country]
    sizes = [t.shape[0] for t in tables]
    padded = [_round8(s) for s in sizes]
    offs = []
    acc = 0
    for p in padded:
        offs.append(acc)
        acc += p
    RP = acc

    tables_p = [t if t.shape[0] == p else jnp.pad(t, ((0, p - t.shape[0]), (0, 0)))
                for t, p in zip(tables, padded)]

    def _proj_body(eh, ehc, ea, eac, et, eci, eco, wh, wa, wi, out_ref):
        regions = [
            (eh,  wh,  CH),
            (ehc, wh,  CH + E),
            (ea,  wa,  CH),
            (eac, wa,  CH + E),
            (et,  wi,  CI),
            (eci, wi,  CI + E),
            (eco, wi,  CI + 2 * E),
        ]
        for k, (eref, wref, ws) in enumerate(regions):
            out_ref[pl.ds(offs[k], padded[k]), :] = jnp.dot(
                eref[...], wref[pl.ds(ws, E), :],
                preferred_element_type=jnp.float32)

    ptab = pl.pallas_call(
        _proj_body,
        out_shape=jax.ShapeDtypeStruct((RP, D), jnp.float32),
    )(*tables_p, W_home, W_away, W_info)

    gidx = jnp.stack([
        home_cate_feat[:, 0] + offs[0],
        home_cate_feat[:, 1] + offs[1],
        away_cate_feat[:, 0] + offs[2],
        away_cate_feat[:, 1] + offs[3],
        info_cate_feat[:, 0] + offs[4],
        info_cate_feat[:, 1] + offs[5],
        info_cate_feat[:, 2] + offs[6],
    ], axis=0).astype(jnp.int32)
    nchunks = B // CHUNK
    idx_chunks = gidx.reshape(7, nchunks, CHUNK).transpose(1, 0, 2)

    nsub = nchunks // NW

    mesh = plsc.VectorSubcoreMesh(core_axis_name="c", subcore_axis_name="s",
                                  num_cores=NC, num_subcores=NS)

    @functools.partial(
        pl.kernel,
        out_type=(jax.ShapeDtypeStruct((B, D), jnp.float32),) * 3,
        mesh=mesh,
        scratch_types=[
            pltpu.VMEM((nsub, 7, CHUNK), jnp.int32),
            [[pltpu.VMEM((CHUNK, D), jnp.float32) for _ in range(7)]
             for _ in range(2)],
            pltpu.SemaphoreType.DMA, pltpu.SemaphoreType.DMA,
            pltpu.SemaphoreType.DMA, pltpu.SemaphoreType.DMA,
        ],
    )
    def _gather_sum(ptab_hbm, idx_hbm, gh_out, ga_out, gi_out,
                    idx_v, bufs, sg0, sg1, ss0, ss1):
        wid = lax.axis_index("s") * NC + lax.axis_index("c")
        first = wid * nsub
        pltpu.sync_copy(idx_hbm.at[pl.ds(first, nsub)], idx_v)
        sem_g = (sg0, sg1)
        sem_s = (ss0, ss1)

        def fire_gathers(s, p):
            return [pltpu.async_copy(ptab_hbm.at[idx_v.at[s, t]],
                                     bufs[p][t], sem_g[p])
                    for t in range(7)]

        gdescs = {0: fire_gathers(0, 0)}
        scat_pending = {0: [], 1: []}
        for s in range(nsub):
            p = s % 2
            q = (s + 1) % 2
            if s + 1 < nsub:
                for dsc in scat_pending[q]:
                    dsc.wait()
                scat_pending[q] = []
                gdescs[q] = fire_gathers(s + 1, q)
            for dsc in gdescs[p]:
                dsc.wait()

            bp = bufs[p]

            def addrow(r, c, bp=bp):
                for cc in range(D // LANES):
                    sl = pl.ds(cc * LANES, LANES)
                    plsc.addupdate(bp[0].at[r, sl], bp[1][r, sl])
                    plsc.addupdate(bp[2].at[r, sl], bp[3][r, sl])
                    plsc.addupdate(bp[4].at[r, sl],
                                   bp[5][r, sl] + bp[6][r, sl])
                return c

            lax.fori_loop(0, CHUNK, addrow, 0, unroll=False)
            base = (first + s) * CHUNK
            scat_pending[p] = [
                pltpu.async_copy(bp[0], gh_out.at[pl.ds(base, CHUNK)], sem_s[p]),
                pltpu.async_copy(bp[2], ga_out.at[pl.ds(base, CHUNK)], sem_s[p]),
                pltpu.async_copy(bp[4], gi_out.at[pl.ds(base, CHUNK)], sem_s[p]),
            ]
        for p in (0, 1):
            for dsc in scat_pending[p]:
                dsc.wait()

    g_home, g_away, g_info = _gather_sum(ptab, idx_chunks)

    BM = 1024
    grid = (B // BM,)

    def _mlp_body(hc, ac, ic, gh, ga, gi, wh, wa, wi, bh, ba, bi,
                  wj, bj, wf, bf, wo, bo, out_ref):
        h = jnp.maximum(
            jnp.dot(hc[...], wh[pl.ds(0, CH), :],
                    preferred_element_type=jnp.float32) + gh[...] + bh[...], 0.0)
        a = jnp.maximum(
            jnp.dot(ac[...], wa[pl.ds(0, CH), :],
                    preferred_element_type=jnp.float32) + ga[...] + ba[...], 0.0)
        i = jnp.maximum(
            jnp.dot(ic[...], wi[pl.ds(0, CI), :],
                    preferred_element_type=jnp.float32) + gi[...] + bi[...], 0.0)
        j = jnp.maximum(
            jnp.dot(h, wj[pl.ds(0, D), :], preferred_element_type=jnp.float32)
            + jnp.dot(a, wj[pl.ds(D, D), :], preferred_element_type=jnp.float32)
            + jnp.dot(i, wj[pl.ds(2 * D, D), :], preferred_element_type=jnp.float32)
            + bj[...], 0.0)
        f = jnp.maximum(
            jnp.dot(j, wf[...], preferred_element_type=jnp.float32) + bf[...], 0.0)
        logit = jnp.sum(f * wo[...], axis=1, keepdims=True) + bo[...]
        out_ref[...] = jax.nn.sigmoid(logit)

    def _rows(cols):
        return pl.BlockSpec((BM, cols), lambda i: (i, 0))

    def _whole(shape):
        return pl.BlockSpec(shape, lambda i: (0, 0))

    out = pl.pallas_call(
        _mlp_body,
        grid=grid,
        in_specs=[
            _rows(CH), _rows(CH), _rows(CI),
            _rows(D), _rows(D), _rows(D),
            _whole(W_home.shape), _whole(W_away.shape), _whole(W_info.shape),
            _whole((1, D)), _whole((1, D)), _whole((1, D)),
            _whole(W_joint.shape), _whole((1, D)),
            _whole(W_fc1.shape), _whole((1, D)),
            _whole((1, D)), _whole((1, 1)),
        ],
        out_specs=pl.BlockSpec((BM, 1), lambda i: (i, 0)),
        out_shape=jax.ShapeDtypeStruct((B, 1), jnp.float32),
        compiler_params=pltpu.CompilerParams(
            dimension_semantics=("arbitrary",)),
    )(home_cont_feat, away_cont_feat, info_cont_feat,
      g_home, g_away, g_info,
      W_home, W_away, W_info,
      b_home.reshape(1, D), b_away.reshape(1, D), b_info.reshape(1, D),
      W_joint, b_joint.reshape(1, D),
      W_fc1, b_fc1.reshape(1, D),
      W_out.reshape(1, D), b_out.reshape(1, 1))
    return out
